# baseline (speedup 1.0000x reference)
import functools

import jax
import jax.numpy as jnp
from jax.experimental import pallas as pl
from jax.experimental.pallas import tpu as pltpu

D_MODEL = 512
DF = 128
LN_EPS = 1e-5

BB = 8
TB = 512
CH = 64


def _detector_kernel(x_ref, m_ref, vcol_ref, bias_ref, w1_ref, b1_ref,
                     gamma_ref, beta_ref, w2_ref, b2_ref, o_ref,
                     s1e, s1l, s2, cnt, mxv, pk, thr_s, *,
                     t_blocks, n_early, t_total):
    phase = pl.program_id(1)
    ti = pl.program_id(2)
    n_ch = TB // CH

    @pl.when(phase == 0)
    def _accumulate():
        @pl.when(ti == 0)
        def _init():
            s1e[...] = jnp.zeros_like(s1e)
            s1l[...] = jnp.zeros_like(s1l)
            s2[...] = jnp.zeros_like(s2)
            cnt[...] = jnp.zeros_like(cnt)
            mxv[...] = jnp.full_like(mxv, -2e6)

        cnt[...] += jnp.sum(1.0 - m_ref[...], axis=1, keepdims=True)

        for b in range(BB):
            acc1 = jnp.zeros((8, D_MODEL), jnp.float32)
            acc2 = jnp.zeros((8, D_MODEL), jnp.float32)
            accm = jnp.full((8, D_MODEL), -2e6, jnp.float32)
            for c in range(n_ch):
                lo = c * CH
                fl = b * TB + lo
                xs = x_ref[b, lo:lo + CH, :]
                vc = vcol_ref[0, 0, fl:fl + CH, :]
                bc = bias_ref[0, 0, fl:fl + CH, :]
                xv = xs * vc
                acc1 = acc1 + jnp.sum(xv.reshape(CH // 8, 8, D_MODEL), axis=0)
                acc2 = acc2 + jnp.sum((xv * xs).reshape(CH // 8, 8, D_MODEL),
                                      axis=0)
                accm = jnp.maximum(
                    accm, jnp.max((xs + bc).reshape(CH // 8, 8, D_MODEL),
                                  axis=0))

            @pl.when(ti < n_early)
            def _():
                s1e[b] += acc1

            @pl.when(ti >= n_early)
            def _():
                s1l[b] += acc1

            s2[b] += acc2
            mxv[b] = jnp.maximum(mxv[b], accm)

    @pl.when(phase == 1)
    def _peaks():
        @pl.when(ti == 0)
        def _init():
            pk[...] = jnp.zeros_like(pk)
            denom = cnt[...] + 1e-8
            pooled = (jnp.sum(s1e[...], axis=1) + jnp.sum(s1l[...], axis=1)
                      ) / denom
            thr_s[...] = jnp.max(pooled, axis=-1, keepdims=True) * 0.7

        for b in range(BB):
            accp = jnp.zeros((8, D_MODEL), jnp.float32)
            thr_b = thr_s[b:b + 1, 0:1]
            for c in range(n_ch):
                lo = c * CH
                fl = b * TB + lo
                xs = x_ref[b, lo:lo + CH, :]
                vc = vcol_ref[0, 0, fl:fl + CH, :]
                hv = jnp.where(xs > thr_b, vc, 0.0)
                accp = accp + jnp.sum(hv.reshape(CH // 8, 8, D_MODEL), axis=0)
            pk[b] += accp

        @pl.when(ti == t_blocks - 1)
        def _head():
            cntv = cnt[...]
            denom = cntv + 1e-8
            s1e_f = jnp.sum(s1e[...], axis=1)
            s1l_f = jnp.sum(s1l[...], axis=1)
            s1 = s1e_f + s1l_f
            pooled = s1 / denom
            m_raw = jnp.max(jnp.max(mxv[...], axis=1), axis=-1,
                            keepdims=True)
            max_strength = jnp.where(cntv < t_total,
                                     jnp.maximum(m_raw, -65000.0), m_raw)
            s2_f = jnp.sum(s2[...], axis=1)
            x_var = (s2_f - 2.0 * pooled * s1
                     + pooled * pooled * cntv) / denom
            variance = jnp.max(x_var, axis=-1, keepdims=True)
            peak_count = jnp.max(jnp.sum(pk[...], axis=1), axis=-1,
                                 keepdims=True)
            early = jnp.max(s1e_f, axis=-1, keepdims=True)
            late = jnp.max(s1l_f, axis=-1, keepdims=True)
            asymmetry = jnp.abs(early - late)
            features = jnp.concatenate(
                [max_strength, variance, peak_count, asymmetry], axis=-1)
            h = jnp.dot(features, w1_ref[...],
                        preferred_element_type=jnp.float32) + b1_ref[...]
            mu = jnp.mean(h, axis=-1, keepdims=True)
            var = jnp.mean((h - mu) ** 2, axis=-1, keepdims=True)
            h = (h - mu) / jnp.sqrt(var + LN_EPS) * gamma_ref[...] + beta_ref[...]
            h = 0.5 * h * (1.0 + jax.lax.erf(h * 0.7071067811865476))
            o_ref[...] = jnp.dot(h, w2_ref[...],
                                 preferred_element_type=jnp.float32) + b2_ref[...]


def kernel(x, padding_mask, W1, b1, gamma, beta, W2, b2):
    B, T, D = x.shape
    t_blocks = T // TB
    b_blocks = B // BB
    n_early = (T // 2) // TB
    maskf = padding_mask.astype(jnp.float32)

    def cols(a):
        return (a.reshape(b_blocks, BB, t_blocks, TB).transpose(0, 2, 1, 3)
                .reshape(b_blocks, t_blocks, BB * TB, 1))

    vcol = cols(1.0 - maskf)
    bias = cols(maskf * -1e6)

    body = functools.partial(_detector_kernel, t_blocks=t_blocks,
                             n_early=n_early, t_total=float(T))
    out = pl.pallas_call(
        body,
        out_shape=jax.ShapeDtypeStruct((B, DF), jnp.float32),
        grid=(b_blocks, 2, t_blocks),
        in_specs=[
            pl.BlockSpec((BB, TB, D), lambda bi, ph, ti: (bi, ti, 0)),
            pl.BlockSpec((BB, TB), lambda bi, ph, ti: (bi, ti)),
            pl.BlockSpec((1, 1, BB * TB, 1), lambda bi, ph, ti: (bi, ti, 0, 0)),
            pl.BlockSpec((1, 1, BB * TB, 1), lambda bi, ph, ti: (bi, ti, 0, 0)),
            pl.BlockSpec((4, DF), lambda bi, ph, ti: (0, 0)),
            pl.BlockSpec((1, DF), lambda bi, ph, ti: (0, 0)),
            pl.BlockSpec((1, DF), lambda bi, ph, ti: (0, 0)),
            pl.BlockSpec((1, DF), lambda bi, ph, ti: (0, 0)),
            pl.BlockSpec((DF, DF), lambda bi, ph, ti: (0, 0)),
            pl.BlockSpec((1, DF), lambda bi, ph, ti: (0, 0)),
        ],
        out_specs=pl.BlockSpec((BB, DF), lambda bi, ph, ti: (bi, 0)),
        scratch_shapes=[
            pltpu.VMEM((BB, 8, D), jnp.float32),
            pltpu.VMEM((BB, 8, D), jnp.float32),
            pltpu.VMEM((BB, 8, D), jnp.float32),
            pltpu.VMEM((BB, 1), jnp.float32),
            pltpu.VMEM((BB, 8, D), jnp.float32),
            pltpu.VMEM((BB, 8, D), jnp.float32),
            pltpu.VMEM((BB, 1), jnp.float32),
        ],
        compiler_params=pltpu.CompilerParams(
            dimension_semantics=("parallel", "arbitrary", "arbitrary"),
            vmem_limit_bytes=56 * 1024 * 1024,
        ),
        name="caustic_detector",
    )(x, maskf, vcol, bias, W1, b1.reshape(1, DF), gamma.reshape(1, DF),
      beta.reshape(1, DF), W2, b2.reshape(1, DF))
    return out


# speedup vs baseline: 1.6640x; 1.6640x over previous
import functools

import jax
import jax.numpy as jnp
from jax.experimental import pallas as pl
from jax.experimental.pallas import tpu as pltpu

D_MODEL = 512
DF = 128
LN_EPS = 1e-5

BB = 8
TB = 512


def _detector_kernel(x_ref, m_ref, w1_ref, b1_ref, gamma_ref, beta_ref,
                     w2_ref, b2_ref, o_ref,
                     s1e, s1l, s2, cnt, mxv, pk, thr_s, *,
                     t_blocks, n_early, t_total):
    phase = pl.program_id(1)
    ti = pl.program_id(2)

    xb = x_ref[...]
    valid2d = 1.0 - m_ref[...]
    valid3 = valid2d[:, :, None]

    @pl.when(phase == 0)
    def _accumulate():
        @pl.when(ti == 0)
        def _init():
            s1e[...] = jnp.zeros_like(s1e)
            s1l[...] = jnp.zeros_like(s1l)
            s2[...] = jnp.zeros_like(s2)
            cnt[...] = jnp.zeros_like(cnt)
            mxv[...] = jnp.full_like(mxv, -2e6)

        xv = xb * valid3
        s1_blk = jnp.sum(xv, axis=1)

        @pl.when(ti < n_early)
        def _():
            s1e[...] += s1_blk

        @pl.when(ti >= n_early)
        def _():
            s1l[...] += s1_blk

        s2[...] += jnp.sum(xv * xb, axis=1)
        cnt[...] += jnp.sum(valid2d, axis=1, keepdims=True)
        bias3 = (valid2d - 1.0)[:, :, None] * 1e6
        mxv[...] = jnp.maximum(mxv[...], jnp.max(xb + bias3, axis=1))

    @pl.when(phase == 1)
    def _peaks():
        @pl.when(ti == 0)
        def _init():
            pk[...] = jnp.zeros_like(pk)
            denom = cnt[...] + 1e-8
            pooled = (s1e[...] + s1l[...]) / denom
            thr_s[...] = jnp.max(pooled, axis=-1, keepdims=True) * 0.7

        thr3 = thr_s[...][:, :, None]
        hv = jnp.where(xb > thr3, valid3, 0.0)
        pk[...] += jnp.sum(hv, axis=1)

        @pl.when(ti == t_blocks - 1)
        def _head():
            cntv = cnt[...]
            denom = cntv + 1e-8
            s1 = s1e[...] + s1l[...]
            pooled = s1 / denom
            m_raw = jnp.max(mxv[...], axis=-1, keepdims=True)
            max_strength = jnp.where(cntv < t_total,
                                     jnp.maximum(m_raw, -65000.0), m_raw)
            x_var = (s2[...] - 2.0 * pooled * s1
                     + pooled * pooled * cntv) / denom
            variance = jnp.max(x_var, axis=-1, keepdims=True)
            peak_count = jnp.max(pk[...], axis=-1, keepdims=True)
            early = jnp.max(s1e[...], axis=-1, keepdims=True)
            late = jnp.max(s1l[...], axis=-1, keepdims=True)
            asymmetry = jnp.abs(early - late)
            features = jnp.concatenate(
                [max_strength, variance, peak_count, asymmetry], axis=-1)
            h = jnp.dot(features, w1_ref[...],
                        preferred_element_type=jnp.float32) + b1_ref[...]
            mu = jnp.mean(h, axis=-1, keepdims=True)
            var = jnp.mean((h - mu) ** 2, axis=-1, keepdims=True)
            h = (h - mu) / jnp.sqrt(var + LN_EPS) * gamma_ref[...] + beta_ref[...]
            h = 0.5 * h * (1.0 + jax.lax.erf(h * 0.7071067811865476))
            o_ref[...] = jnp.dot(h, w2_ref[...],
                                 preferred_element_type=jnp.float32) + b2_ref[...]


def kernel(x, padding_mask, W1, b1, gamma, beta, W2, b2):
    B, T, D = x.shape
    t_blocks = T // TB
    n_early = (T // 2) // TB
    maskf = padding_mask.astype(jnp.float32)

    body = functools.partial(_detector_kernel, t_blocks=t_blocks,
                             n_early=n_early, t_total=float(T))
    out = pl.pallas_call(
        body,
        out_shape=jax.ShapeDtypeStruct((B, DF), jnp.float32),
        grid=(B // BB, 2, t_blocks),
        in_specs=[
            pl.BlockSpec((BB, TB, D), lambda bi, ph, ti: (bi, ti, 0)),
            pl.BlockSpec((BB, TB), lambda bi, ph, ti: (bi, ti)),
            pl.BlockSpec((4, DF), lambda bi, ph, ti: (0, 0)),
            pl.BlockSpec((1, DF), lambda bi, ph, ti: (0, 0)),
            pl.BlockSpec((1, DF), lambda bi, ph, ti: (0, 0)),
            pl.BlockSpec((1, DF), lambda bi, ph, ti: (0, 0)),
            pl.BlockSpec((DF, DF), lambda bi, ph, ti: (0, 0)),
            pl.BlockSpec((1, DF), lambda bi, ph, ti: (0, 0)),
        ],
        out_specs=pl.BlockSpec((BB, DF), lambda bi, ph, ti: (bi, 0)),
        scratch_shapes=[
            pltpu.VMEM((BB, D), jnp.float32),
            pltpu.VMEM((BB, D), jnp.float32),
            pltpu.VMEM((BB, D), jnp.float32),
            pltpu.VMEM((BB, 1), jnp.float32),
            pltpu.VMEM((BB, D), jnp.float32),
            pltpu.VMEM((BB, D), jnp.float32),
            pltpu.VMEM((BB, 1), jnp.float32),
        ],
        compiler_params=pltpu.CompilerParams(
            dimension_semantics=("parallel", "arbitrary", "arbitrary"),
            vmem_limit_bytes=56 * 1024 * 1024,
        ),
        name="caustic_detector",
    )(x, maskf, W1, b1.reshape(1, DF), gamma.reshape(1, DF),
      beta.reshape(1, DF), W2, b2.reshape(1, DF))
    return out


# speedup vs baseline: 1.7965x; 1.0796x over previous
import functools

import jax
import jax.numpy as jnp
from jax.experimental import pallas as pl
from jax.experimental.pallas import tpu as pltpu

D_MODEL = 512
DF = 128
LN_EPS = 1e-5

BB = 16
TB = 256


def _detector_kernel(x_ref, m_ref, w1_ref, b1_ref, gamma_ref, beta_ref,
                     w2_ref, b2_ref, o_ref,
                     s1e, s1l, s2, cnt, mx, pk, *, t_blocks, n_early):
    phase = pl.program_id(1)
    ti = pl.program_id(2)

    xb = x_ref[...]
    valid = 1.0 - m_ref[...]

    @pl.when(phase == 0)
    def _accumulate():
        @pl.when(ti == 0)
        def _init():
            s1e[...] = jnp.zeros_like(s1e)
            s1l[...] = jnp.zeros_like(s1l)
            s2[...] = jnp.zeros_like(s2)
            cnt[...] = jnp.zeros_like(cnt)
            mx[...] = jnp.full_like(mx, -65000.0)

        xv = xb * valid[:, :, None]
        s1_blk = jnp.sum(xv, axis=1)

        @pl.when(ti < n_early)
        def _():
            s1e[...] += s1_blk

        @pl.when(ti >= n_early)
        def _():
            s1l[...] += s1_blk

        s2[...] += jnp.sum(xv * xb, axis=1)
        cnt[...] += jnp.sum(valid, axis=1, keepdims=True)
        x_masked = jnp.where(m_ref[...][:, :, None] > 0.0, -65000.0, xb)
        mx[...] = jnp.maximum(mx[...],
                              jnp.max(x_masked, axis=(1, 2), keepdims=False)[:, None])

    @pl.when(phase == 1)
    def _peaks():
        denom = cnt[...] + 1e-8
        s1 = s1e[...] + s1l[...]
        pooled = s1 / denom
        thr = jnp.max(pooled, axis=-1, keepdims=True) * 0.7

        @pl.when(ti == 0)
        def _init():
            pk[...] = jnp.zeros_like(pk)

        high = (xb > thr[:, :, None]).astype(jnp.float32)
        pk[...] += jnp.sum(high * valid[:, :, None], axis=1)

        @pl.when(ti == t_blocks - 1)
        def _head():
            max_strength = mx[...]
            x_var = (s2[...] - 2.0 * pooled * s1
                     + pooled * pooled * cnt[...]) / denom
            variance = jnp.max(x_var, axis=-1, keepdims=True)
            peak_count = jnp.max(pk[...], axis=-1, keepdims=True)
            early = jnp.max(s1e[...], axis=-1, keepdims=True)
            late = jnp.max(s1l[...], axis=-1, keepdims=True)
            asymmetry = jnp.abs(early - late)
            features = jnp.concatenate(
                [max_strength, variance, peak_count, asymmetry], axis=-1)
            h = jnp.dot(features, w1_ref[...],
                        preferred_element_type=jnp.float32) + b1_ref[...]
            mu = jnp.mean(h, axis=-1, keepdims=True)
            var = jnp.mean((h - mu) ** 2, axis=-1, keepdims=True)
            h = (h - mu) / jnp.sqrt(var + LN_EPS) * gamma_ref[...] + beta_ref[...]
            h = 0.5 * h * (1.0 + jax.lax.erf(h * 0.7071067811865476))
            o_ref[...] = jnp.dot(h, w2_ref[...],
                                 preferred_element_type=jnp.float32) + b2_ref[...]


def kernel(x, padding_mask, W1, b1, gamma, beta, W2, b2):
    B, T, D = x.shape
    t_blocks = T // TB
    n_early = (T // 2) // TB
    maskf = padding_mask.astype(jnp.float32)

    body = functools.partial(_detector_kernel, t_blocks=t_blocks,
                             n_early=n_early)
    out = pl.pallas_call(
        body,
        out_shape=jax.ShapeDtypeStruct((B, DF), jnp.float32),
        grid=(B // BB, 2, t_blocks),
        in_specs=[
            pl.BlockSpec((BB, TB, D), lambda bi, ph, ti: (bi, ti, 0)),
            pl.BlockSpec((BB, TB), lambda bi, ph, ti: (bi, ti)),
            pl.BlockSpec((4, DF), lambda bi, ph, ti: (0, 0)),
            pl.BlockSpec((1, DF), lambda bi, ph, ti: (0, 0)),
            pl.BlockSpec((1, DF), lambda bi, ph, ti: (0, 0)),
            pl.BlockSpec((1, DF), lambda bi, ph, ti: (0, 0)),
            pl.BlockSpec((DF, DF), lambda bi, ph, ti: (0, 0)),
            pl.BlockSpec((1, DF), lambda bi, ph, ti: (0, 0)),
        ],
        out_specs=pl.BlockSpec((BB, DF), lambda bi, ph, ti: (bi, 0)),
        scratch_shapes=[
            pltpu.VMEM((BB, D), jnp.float32),
            pltpu.VMEM((BB, D), jnp.float32),
            pltpu.VMEM((BB, D), jnp.float32),
            pltpu.VMEM((BB, 1), jnp.float32),
            pltpu.VMEM((BB, 1), jnp.float32),
            pltpu.VMEM((BB, D), jnp.float32),
        ],
        compiler_params=pltpu.CompilerParams(
            dimension_semantics=("parallel", "arbitrary", "arbitrary"),
            vmem_limit_bytes=56 * 1024 * 1024,
        ),
        name="caustic_detector",
    )(x, maskf, W1, b1.reshape(1, DF), gamma.reshape(1, DF),
      beta.reshape(1, DF), W2, b2.reshape(1, DF))
    return out


# speedup vs baseline: 1.8353x; 1.0216x over previous
import functools

import jax
import jax.numpy as jnp
from jax.experimental import pallas as pl
from jax.experimental.pallas import tpu as pltpu

D_MODEL = 512
DF = 128
LN_EPS = 1e-5

BB = 8
TB = 512


def _detector_kernel(x_ref, m_ref, w1_ref, b1_ref, gamma_ref, beta_ref,
                     w2_ref, b2_ref, o_ref,
                     s1e, s1l, s2, cnt, mx, pk, *, t_blocks, n_early):
    phase = pl.program_id(1)
    ti = pl.program_id(2)

    xb = x_ref[...]
    valid = 1.0 - m_ref[...]

    @pl.when(phase == 0)
    def _accumulate():
        @pl.when(ti == 0)
        def _init():
            s1e[...] = jnp.zeros_like(s1e)
            s1l[...] = jnp.zeros_like(s1l)
            s2[...] = jnp.zeros_like(s2)
            cnt[...] = jnp.zeros_like(cnt)
            mx[...] = jnp.full_like(mx, -65000.0)

        xv = xb * valid[:, :, None]
        s1_blk = jnp.sum(xv, axis=1)

        @pl.when(ti < n_early)
        def _():
            s1e[...] += s1_blk

        @pl.when(ti >= n_early)
        def _():
            s1l[...] += s1_blk

        s2[...] += jnp.sum(xv * xb, axis=1)
        cnt[...] += jnp.sum(valid, axis=1, keepdims=True)
        x_masked = jnp.where(m_ref[...][:, :, None] > 0.0, -65000.0, xb)
        mx[...] = jnp.maximum(mx[...],
                              jnp.max(x_masked, axis=(1, 2), keepdims=False)[:, None])

    @pl.when(phase == 1)
    def _peaks():
        denom = cnt[...] + 1e-8
        s1 = s1e[...] + s1l[...]
        pooled = s1 / denom
        thr = jnp.max(pooled, axis=-1, keepdims=True) * 0.7

        @pl.when(ti == 0)
        def _init():
            pk[...] = jnp.zeros_like(pk)

        high = (xb > thr[:, :, None]).astype(jnp.float32)
        pk[...] += jnp.sum(high * valid[:, :, None], axis=1)

        @pl.when(ti == t_blocks - 1)
        def _head():
            max_strength = mx[...]
            x_var = (s2[...] - 2.0 * pooled * s1
                     + pooled * pooled * cnt[...]) / denom
            variance = jnp.max(x_var, axis=-1, keepdims=True)
            peak_count = jnp.max(pk[...], axis=-1, keepdims=True)
            early = jnp.max(s1e[...], axis=-1, keepdims=True)
            late = jnp.max(s1l[...], axis=-1, keepdims=True)
            asymmetry = jnp.abs(early - late)
            features = jnp.concatenate(
                [max_strength, variance, peak_count, asymmetry], axis=-1)
            h = jnp.dot(features, w1_ref[...],
                        preferred_element_type=jnp.float32) + b1_ref[...]
            mu = jnp.mean(h, axis=-1, keepdims=True)
            var = jnp.mean((h - mu) ** 2, axis=-1, keepdims=True)
            h = (h - mu) / jnp.sqrt(var + LN_EPS) * gamma_ref[...] + beta_ref[...]
            h = 0.5 * h * (1.0 + jax.lax.erf(h * 0.7071067811865476))
            o_ref[...] = jnp.dot(h, w2_ref[...],
                                 preferred_element_type=jnp.float32) + b2_ref[...]


def kernel(x, padding_mask, W1, b1, gamma, beta, W2, b2):
    B, T, D = x.shape
    t_blocks = T // TB
    n_early = (T // 2) // TB
    maskf = padding_mask.astype(jnp.float32)

    body = functools.partial(_detector_kernel, t_blocks=t_blocks,
                             n_early=n_early)
    out = pl.pallas_call(
        body,
        out_shape=jax.ShapeDtypeStruct((B, DF), jnp.float32),
        grid=(B // BB, 2, t_blocks),
        in_specs=[
            pl.BlockSpec((BB, TB, D), lambda bi, ph, ti: (bi, ti, 0)),
            pl.BlockSpec((BB, TB), lambda bi, ph, ti: (bi, ti)),
            pl.BlockSpec((4, DF), lambda bi, ph, ti: (0, 0)),
            pl.BlockSpec((1, DF), lambda bi, ph, ti: (0, 0)),
            pl.BlockSpec((1, DF), lambda bi, ph, ti: (0, 0)),
            pl.BlockSpec((1, DF), lambda bi, ph, ti: (0, 0)),
            pl.BlockSpec((DF, DF), lambda bi, ph, ti: (0, 0)),
            pl.BlockSpec((1, DF), lambda bi, ph, ti: (0, 0)),
        ],
        out_specs=pl.BlockSpec((BB, DF), lambda bi, ph, ti: (bi, 0)),
        scratch_shapes=[
            pltpu.VMEM((BB, D), jnp.float32),
            pltpu.VMEM((BB, D), jnp.float32),
            pltpu.VMEM((BB, D), jnp.float32),
            pltpu.VMEM((BB, 1), jnp.float32),
            pltpu.VMEM((BB, 1), jnp.float32),
            pltpu.VMEM((BB, D), jnp.float32),
        ],
        compiler_params=pltpu.CompilerParams(
            dimension_semantics=("parallel", "arbitrary", "arbitrary"),
            vmem_limit_bytes=56 * 1024 * 1024,
        ),
        name="caustic_detector",
    )(x, maskf, W1, b1.reshape(1, DF), gamma.reshape(1, DF),
      beta.reshape(1, DF), W2, b2.reshape(1, DF))
    return out


# speedup vs baseline: 1.8458x; 1.0057x over previous
import functools

import jax
import jax.numpy as jnp
from jax.experimental import pallas as pl
from jax.experimental.pallas import tpu as pltpu

D_MODEL = 512
DF = 128
LN_EPS = 1e-5

BB = 8
TB = 512


def _detector_kernel(x_ref, m_ref, w1_ref, b1_ref, gamma_ref, beta_ref,
                     w2_ref, b2_ref, o_ref,
                     s1e, s1l, s2, cnt, mx, pk, thr_s, *, t_blocks, n_early):
    phase = pl.program_id(1)
    ti = pl.program_id(2)

    xb = x_ref[...]
    valid = 1.0 - m_ref[...]

    @pl.when(phase == 0)
    def _accumulate():
        @pl.when(ti == 0)
        def _init():
            s1e[...] = jnp.zeros_like(s1e)
            s1l[...] = jnp.zeros_like(s1l)
            s2[...] = jnp.zeros_like(s2)
            cnt[...] = jnp.zeros_like(cnt)
            mx[...] = jnp.full_like(mx, -65000.0)

        xv = xb * valid[:, :, None]
        s1_blk = jnp.sum(xv, axis=1)

        @pl.when(ti < n_early)
        def _():
            s1e[...] += s1_blk

        @pl.when(ti >= n_early)
        def _():
            s1l[...] += s1_blk

        s2[...] += jnp.sum(xv * xb, axis=1)
        cnt[...] += jnp.sum(valid, axis=1, keepdims=True)
        x_masked = jnp.where(m_ref[...][:, :, None] > 0.0, -65000.0, xb)
        mx[...] = jnp.maximum(mx[...],
                              jnp.max(x_masked, axis=(1, 2), keepdims=False)[:, None])

    @pl.when(phase == 1)
    def _peaks():
        @pl.when(ti == 0)
        def _init():
            pk[...] = jnp.zeros_like(pk)
            denom0 = cnt[...] + 1e-8
            pooled0 = (s1e[...] + s1l[...]) / denom0
            thr_s[...] = jnp.max(pooled0, axis=-1, keepdims=True) * 0.7

        thr = thr_s[...]
        high = (xb > thr[:, :, None]).astype(jnp.float32)
        pk[...] += jnp.sum(high * valid[:, :, None], axis=1)

        @pl.when(ti == t_blocks - 1)
        def _head():
            denom = cnt[...] + 1e-8
            s1 = s1e[...] + s1l[...]
            pooled = s1 / denom
            max_strength = mx[...]
            x_var = (s2[...] - 2.0 * pooled * s1
                     + pooled * pooled * cnt[...]) / denom
            variance = jnp.max(x_var, axis=-1, keepdims=True)
            peak_count = jnp.max(pk[...], axis=-1, keepdims=True)
            early = jnp.max(s1e[...], axis=-1, keepdims=True)
            late = jnp.max(s1l[...], axis=-1, keepdims=True)
            asymmetry = jnp.abs(early - late)
            features = jnp.concatenate(
                [max_strength, variance, peak_count, asymmetry], axis=-1)
            h = jnp.dot(features, w1_ref[...],
                        preferred_element_type=jnp.float32) + b1_ref[...]
            mu = jnp.mean(h, axis=-1, keepdims=True)
            var = jnp.mean((h - mu) ** 2, axis=-1, keepdims=True)
            h = (h - mu) / jnp.sqrt(var + LN_EPS) * gamma_ref[...] + beta_ref[...]
            h = 0.5 * h * (1.0 + jax.lax.erf(h * 0.7071067811865476))
            o_ref[...] = jnp.dot(h, w2_ref[...],
                                 preferred_element_type=jnp.float32) + b2_ref[...]


def kernel(x, padding_mask, W1, b1, gamma, beta, W2, b2):
    B, T, D = x.shape
    t_blocks = T // TB
    n_early = (T // 2) // TB
    maskf = padding_mask.astype(jnp.float32)

    body = functools.partial(_detector_kernel, t_blocks=t_blocks,
                             n_early=n_early)
    out = pl.pallas_call(
        body,
        out_shape=jax.ShapeDtypeStruct((B, DF), jnp.float32),
        grid=(B // BB, 2, t_blocks),
        in_specs=[
            pl.BlockSpec((BB, TB, D), lambda bi, ph, ti: (bi, ti, 0)),
            pl.BlockSpec((BB, TB), lambda bi, ph, ti: (bi, ti)),
            pl.BlockSpec((4, DF), lambda bi, ph, ti: (0, 0)),
            pl.BlockSpec((1, DF), lambda bi, ph, ti: (0, 0)),
            pl.BlockSpec((1, DF), lambda bi, ph, ti: (0, 0)),
            pl.BlockSpec((1, DF), lambda bi, ph, ti: (0, 0)),
            pl.BlockSpec((DF, DF), lambda bi, ph, ti: (0, 0)),
            pl.BlockSpec((1, DF), lambda bi, ph, ti: (0, 0)),
        ],
        out_specs=pl.BlockSpec((BB, DF), lambda bi, ph, ti: (bi, 0)),
        scratch_shapes=[
            pltpu.VMEM((BB, D), jnp.float32),
            pltpu.VMEM((BB, D), jnp.float32),
            pltpu.VMEM((BB, D), jnp.float32),
            pltpu.VMEM((BB, 1), jnp.float32),
            pltpu.VMEM((BB, 1), jnp.float32),
            pltpu.VMEM((BB, D), jnp.float32),
            pltpu.VMEM((BB, 1), jnp.float32),
        ],
        compiler_params=pltpu.CompilerParams(
            dimension_semantics=("parallel", "arbitrary", "arbitrary"),
            vmem_limit_bytes=56 * 1024 * 1024,
        ),
        name="caustic_detector",
    )(x, maskf, W1, b1.reshape(1, DF), gamma.reshape(1, DF),
      beta.reshape(1, DF), W2, b2.reshape(1, DF))
    return out


# speedup vs baseline: 1.8776x; 1.0172x over previous
import functools

import jax
import jax.numpy as jnp
from jax.experimental import pallas as pl
from jax.experimental.pallas import tpu as pltpu

D_MODEL = 512
DF = 128
LN_EPS = 1e-5

BB = 8
TB = 512


def _detector_kernel(x_ref, m_ref, w1_ref, b1_ref, gamma_ref, beta_ref,
                     w2_ref, b2_ref, o_ref,
                     s1e, s1l, s2, cnt, mx, pk, thr_s, *, t_blocks, n_early):
    phase = pl.program_id(1)
    ti = pl.program_id(2)

    xb = x_ref[...]
    valid = 1.0 - m_ref[...]

    @pl.when(phase == 0)
    def _accumulate():
        @pl.when(ti == 0)
        def _init():
            s1e[...] = jnp.zeros_like(s1e)
            s1l[...] = jnp.zeros_like(s1l)
            s2[...] = jnp.zeros_like(s2)
            cnt[...] = jnp.zeros_like(cnt)
            mx[...] = jnp.full_like(mx, -65000.0)

        xv = xb * valid[:, :, None]
        s1_blk = jnp.sum(xv, axis=1)

        @pl.when(ti < n_early)
        def _():
            s1e[...] += s1_blk

        @pl.when(ti >= n_early)
        def _():
            s1l[...] += s1_blk

        s2[...] += jnp.sum(xv * xb, axis=1)
        cnt[...] += jnp.sum(valid, axis=1, keepdims=True)
        x_masked = jnp.where(m_ref[...][:, :, None] > 0.0, -65000.0, xb)
        mx[...] = jnp.maximum(mx[...],
                              jnp.max(x_masked, axis=(1, 2), keepdims=False)[:, None])

    @pl.when(phase == 1)
    def _peaks():
        @pl.when(ti == 0)
        def _init():
            pk[...] = jnp.zeros_like(pk)
            denom0 = cnt[...] + 1e-8
            pooled0 = (s1e[...] + s1l[...]) / denom0
            thr_s[...] = jnp.max(pooled0, axis=-1, keepdims=True) * 0.7

        thr = thr_s[...]
        hv = jnp.where(xb > thr[:, :, None], valid[:, :, None], 0.0)
        pk[...] += jnp.sum(hv, axis=1)

        @pl.when(ti == t_blocks - 1)
        def _head():
            denom = cnt[...] + 1e-8
            s1 = s1e[...] + s1l[...]
            pooled = s1 / denom
            max_strength = mx[...]
            x_var = (s2[...] - 2.0 * pooled * s1
                     + pooled * pooled * cnt[...]) / denom
            variance = jnp.max(x_var, axis=-1, keepdims=True)
            peak_count = jnp.max(pk[...], axis=-1, keepdims=True)
            early = jnp.max(s1e[...], axis=-1, keepdims=True)
            late = jnp.max(s1l[...], axis=-1, keepdims=True)
            asymmetry = jnp.abs(early - late)
            features = jnp.concatenate(
                [max_strength, variance, peak_count, asymmetry], axis=-1)
            h = jnp.dot(features, w1_ref[...],
                        preferred_element_type=jnp.float32) + b1_ref[...]
            mu = jnp.mean(h, axis=-1, keepdims=True)
            var = jnp.mean((h - mu) ** 2, axis=-1, keepdims=True)
            h = (h - mu) / jnp.sqrt(var + LN_EPS) * gamma_ref[...] + beta_ref[...]
            h = 0.5 * h * (1.0 + jax.lax.erf(h * 0.7071067811865476))
            o_ref[...] = jnp.dot(h, w2_ref[...],
                                 preferred_element_type=jnp.float32) + b2_ref[...]


def kernel(x, padding_mask, W1, b1, gamma, beta, W2, b2):
    B, T, D = x.shape
    t_blocks = T // TB
    n_early = (T // 2) // TB
    maskf = padding_mask.astype(jnp.float32)

    body = functools.partial(_detector_kernel, t_blocks=t_blocks,
                             n_early=n_early)
    out = pl.pallas_call(
        body,
        out_shape=jax.ShapeDtypeStruct((B, DF), jnp.float32),
        grid=(B // BB, 2, t_blocks),
        in_specs=[
            pl.BlockSpec((BB, TB, D), lambda bi, ph, ti: (bi, ti, 0)),
            pl.BlockSpec((BB, TB), lambda bi, ph, ti: (bi, ti)),
            pl.BlockSpec((4, DF), lambda bi, ph, ti: (0, 0)),
            pl.BlockSpec((1, DF), lambda bi, ph, ti: (0, 0)),
            pl.BlockSpec((1, DF), lambda bi, ph, ti: (0, 0)),
            pl.BlockSpec((1, DF), lambda bi, ph, ti: (0, 0)),
            pl.BlockSpec((DF, DF), lambda bi, ph, ti: (0, 0)),
            pl.BlockSpec((1, DF), lambda bi, ph, ti: (0, 0)),
        ],
        out_specs=pl.BlockSpec((BB, DF), lambda bi, ph, ti: (bi, 0)),
        scratch_shapes=[
            pltpu.VMEM((BB, D), jnp.float32),
            pltpu.VMEM((BB, D), jnp.float32),
            pltpu.VMEM((BB, D), jnp.float32),
            pltpu.VMEM((BB, 1), jnp.float32),
            pltpu.VMEM((BB, 1), jnp.float32),
            pltpu.VMEM((BB, D), jnp.float32),
            pltpu.VMEM((BB, 1), jnp.float32),
        ],
        compiler_params=pltpu.CompilerParams(
            dimension_semantics=("parallel", "arbitrary", "arbitrary"),
            vmem_limit_bytes=56 * 1024 * 1024,
        ),
        name="caustic_detector",
    )(x, maskf, W1, b1.reshape(1, DF), gamma.reshape(1, DF),
      beta.reshape(1, DF), W2, b2.reshape(1, DF))
    return out


# speedup vs baseline: 1.8838x; 1.0033x over previous
import functools

import jax
import jax.numpy as jnp
from jax.experimental import pallas as pl
from jax.experimental.pallas import tpu as pltpu

D_MODEL = 512
DF = 128
LN_EPS = 1e-5

BB = 8
TB = 512


def _detector_kernel(x_ref, m_ref, w1_ref, b1_ref, gamma_ref, beta_ref,
                     w2_ref, b2_ref, o_ref,
                     s1e, s1l, s2, cnt, mx, pk, thr_s, *, t_blocks, n_early):
    phase = pl.program_id(1)
    ti = pl.program_id(2)

    xb = x_ref[...]
    valid = 1.0 - m_ref[...]

    @pl.when(phase == 0)
    def _accumulate():
        @pl.when(ti == 0)
        def _init():
            s1e[...] = jnp.zeros_like(s1e)
            s1l[...] = jnp.zeros_like(s1l)
            s2[...] = jnp.zeros_like(s2)
            cnt[...] = jnp.zeros_like(cnt)
            mx[...] = jnp.full_like(mx, -65000.0)

        xv = xb * valid[:, :, None]
        s1_blk = jnp.sum(xv, axis=1)

        @pl.when(ti < n_early)
        def _():
            s1e[...] += s1_blk

        @pl.when(ti >= n_early)
        def _():
            s1l[...] += s1_blk

        s2[...] += jnp.sum(xv * xb, axis=1)
        cnt[...] += jnp.sum(valid, axis=1, keepdims=True)
        x_masked = xb + (valid - 1.0)[:, :, None] * 1e6
        mx[...] = jnp.maximum(mx[...],
                              jnp.max(x_masked, axis=(1, 2), keepdims=False)[:, None])

    @pl.when(phase == 1)
    def _peaks():
        @pl.when(ti == 0)
        def _init():
            pk[...] = jnp.zeros_like(pk)
            denom0 = cnt[...] + 1e-8
            pooled0 = (s1e[...] + s1l[...]) / denom0
            thr_s[...] = jnp.max(pooled0, axis=-1, keepdims=True) * 0.7

        thr = thr_s[...]
        hv = jnp.where(xb > thr[:, :, None], valid[:, :, None], 0.0)
        pk[...] += jnp.sum(hv, axis=1)

        @pl.when(ti == t_blocks - 1)
        def _head():
            denom = cnt[...] + 1e-8
            s1 = s1e[...] + s1l[...]
            pooled = s1 / denom
            max_strength = mx[...]
            x_var = (s2[...] - 2.0 * pooled * s1
                     + pooled * pooled * cnt[...]) / denom
            variance = jnp.max(x_var, axis=-1, keepdims=True)
            peak_count = jnp.max(pk[...], axis=-1, keepdims=True)
            early = jnp.max(s1e[...], axis=-1, keepdims=True)
            late = jnp.max(s1l[...], axis=-1, keepdims=True)
            asymmetry = jnp.abs(early - late)
            features = jnp.concatenate(
                [max_strength, variance, peak_count, asymmetry], axis=-1)
            h = jnp.dot(features, w1_ref[...],
                        preferred_element_type=jnp.float32) + b1_ref[...]
            mu = jnp.mean(h, axis=-1, keepdims=True)
            var = jnp.mean((h - mu) ** 2, axis=-1, keepdims=True)
            h = (h - mu) / jnp.sqrt(var + LN_EPS) * gamma_ref[...] + beta_ref[...]
            h = 0.5 * h * (1.0 + jax.lax.erf(h * 0.7071067811865476))
            o_ref[...] = jnp.dot(h, w2_ref[...],
                                 preferred_element_type=jnp.float32) + b2_ref[...]


def kernel(x, padding_mask, W1, b1, gamma, beta, W2, b2):
    B, T, D = x.shape
    t_blocks = T // TB
    n_early = (T // 2) // TB
    maskf = padding_mask.astype(jnp.float32)

    body = functools.partial(_detector_kernel, t_blocks=t_blocks,
                             n_early=n_early)
    out = pl.pallas_call(
        body,
        out_shape=jax.ShapeDtypeStruct((B, DF), jnp.float32),
        grid=(B // BB, 2, t_blocks),
        in_specs=[
            pl.BlockSpec((BB, TB, D), lambda bi, ph, ti: (bi, ti, 0)),
            pl.BlockSpec((BB, TB), lambda bi, ph, ti: (bi, ti)),
            pl.BlockSpec((4, DF), lambda bi, ph, ti: (0, 0)),
            pl.BlockSpec((1, DF), lambda bi, ph, ti: (0, 0)),
            pl.BlockSpec((1, DF), lambda bi, ph, ti: (0, 0)),
            pl.BlockSpec((1, DF), lambda bi, ph, ti: (0, 0)),
            pl.BlockSpec((DF, DF), lambda bi, ph, ti: (0, 0)),
            pl.BlockSpec((1, DF), lambda bi, ph, ti: (0, 0)),
        ],
        out_specs=pl.BlockSpec((BB, DF), lambda bi, ph, ti: (bi, 0)),
        scratch_shapes=[
            pltpu.VMEM((BB, D), jnp.float32),
            pltpu.VMEM((BB, D), jnp.float32),
            pltpu.VMEM((BB, D), jnp.float32),
            pltpu.VMEM((BB, 1), jnp.float32),
            pltpu.VMEM((BB, 1), jnp.float32),
            pltpu.VMEM((BB, D), jnp.float32),
            pltpu.VMEM((BB, 1), jnp.float32),
        ],
        compiler_params=pltpu.CompilerParams(
            dimension_semantics=("parallel", "arbitrary", "arbitrary"),
            vmem_limit_bytes=56 * 1024 * 1024,
        ),
        name="caustic_detector",
    )(x, maskf, W1, b1.reshape(1, DF), gamma.reshape(1, DF),
      beta.reshape(1, DF), W2, b2.reshape(1, DF))
    return out
